# Initial kernel scaffold; baseline (speedup 1.0000x reference)
import jax
import jax.numpy as jnp
from jax.experimental import pallas as pl


def kernel(tokens, W_E):
    raise NotImplementedError("write your pallas kernel here")


# speedup vs baseline: 1.5720x; 1.5720x over previous
import functools

import jax
import jax.numpy as jnp
from jax import lax
from jax.experimental import pallas as pl
from jax.experimental.pallas import tpu as pltpu
from jax.experimental.pallas import tpu_sc as plsc

_NUM_CORES = 2
_NUM_SUBCORES = 16
_NW = _NUM_CORES * _NUM_SUBCORES
_CHUNK = 64


@functools.lru_cache(maxsize=None)
def _build_embed(vocab, d_model, n_chunks):
    mesh = plsc.VectorSubcoreMesh(core_axis_name="c", subcore_axis_name="s")
    b_per_w = n_chunks * _CHUNK
    batch = _NW * b_per_w

    @functools.partial(
        pl.kernel,
        mesh=mesh,
        out_type=jax.ShapeDtypeStruct((batch, d_model), jnp.float32),
        scratch_types=[
            pltpu.VMEM((n_chunks, _CHUNK), jnp.int32),
            pltpu.VMEM((_CHUNK, d_model), jnp.float32),
            pltpu.VMEM((_CHUNK, d_model), jnp.float32),
            pltpu.SemaphoreType.DMA,
            pltpu.SemaphoreType.DMA,
            pltpu.SemaphoreType.DMA,
            pltpu.SemaphoreType.DMA,
        ],
    )
    def embed(idx_hbm, table_hbm, out_hbm, idx_v, buf0, buf1, sg0, sg1, sw0, sw1):
        wid = lax.axis_index("s") * _NUM_CORES + lax.axis_index("c")
        base = wid * b_per_w
        bufs = (buf0, buf1)
        gsems = (sg0, sg1)
        wsems = (sw0, sw1)

        pltpu.sync_copy(idx_hbm.at[wid], idx_v)

        gathers = [None] * n_chunks
        writes = [None] * n_chunks
        gathers[0] = pltpu.async_copy(
            table_hbm.at[idx_v.at[0]], bufs[0], gsems[0])
        for j in range(n_chunks):
            if j + 1 < n_chunks:
                if j >= 1:
                    writes[j - 1].wait()
                gathers[j + 1] = pltpu.async_copy(
                    table_hbm.at[idx_v.at[j + 1]],
                    bufs[(j + 1) % 2], gsems[(j + 1) % 2])
            gathers[j].wait()
            writes[j] = pltpu.async_copy(
                bufs[j % 2],
                out_hbm.at[pl.ds(base + j * _CHUNK, _CHUNK)],
                wsems[j % 2])
        if n_chunks >= 2:
            writes[n_chunks - 2].wait()
        writes[n_chunks - 1].wait()

    return embed


def kernel(tokens, W_E):
    d_model = W_E.shape[1]
    b = tokens.size
    assert b % (_NW * _CHUNK) == 0
    n_chunks = b // (_NW * _CHUNK)
    idx = tokens.reshape(_NW, n_chunks, _CHUNK).astype(jnp.int32)
    out = _build_embed(W_E.shape[0], d_model, n_chunks)(idx, W_E)
    return out.reshape(*tokens.shape, d_model)


# speedup vs baseline: 1.5884x; 1.0104x over previous
import functools

import jax
import jax.numpy as jnp
from jax import lax
from jax.experimental import pallas as pl
from jax.experimental.pallas import tpu as pltpu
from jax.experimental.pallas import tpu_sc as plsc

_NUM_CORES = 2
_NUM_SUBCORES = 16
_NW = _NUM_CORES * _NUM_SUBCORES
_CHUNK = 32
_NBUF = 4


@functools.lru_cache(maxsize=None)
def _build_embed(vocab, d_model, n_chunks):
    mesh = plsc.VectorSubcoreMesh(core_axis_name="c", subcore_axis_name="s")
    b_per_w = n_chunks * _CHUNK
    batch = _NW * b_per_w

    @functools.partial(
        pl.kernel,
        mesh=mesh,
        out_type=jax.ShapeDtypeStruct((batch, d_model), jnp.float32),
        scratch_types=(
            [pltpu.VMEM((n_chunks, _CHUNK), jnp.int32)]
            + [pltpu.VMEM((_CHUNK, d_model), jnp.float32) for _ in range(_NBUF)]
            + [pltpu.SemaphoreType.DMA for _ in range(2 * _NBUF)]
        ),
    )
    def embed(idx_hbm, table_hbm, out_hbm, idx_v, *rest):
        bufs = rest[:_NBUF]
        gsems = rest[_NBUF:2 * _NBUF]
        wsems = rest[2 * _NBUF:]
        wid = lax.axis_index("s") * _NUM_CORES + lax.axis_index("c")
        base = wid * b_per_w

        pltpu.sync_copy(idx_hbm.at[wid], idx_v)

        def start_gather(j):
            return pltpu.async_copy(
                table_hbm.at[idx_v.at[j]], bufs[j % _NBUF], gsems[j % _NBUF])

        def start_write(j):
            return pltpu.async_copy(
                bufs[j % _NBUF],
                out_hbm.at[pl.ds(base + j * _CHUNK, _CHUNK)],
                wsems[j % _NBUF])

        gathers = [None] * n_chunks
        writes = [None] * n_chunks
        for j in range(min(_NBUF - 1, n_chunks)):
            gathers[j] = start_gather(j)
        for j in range(n_chunks):
            nxt = j + _NBUF - 1
            if nxt < n_chunks:
                if nxt - _NBUF >= 0:
                    writes[nxt - _NBUF].wait()
                gathers[nxt] = start_gather(nxt)
            gathers[j].wait()
            writes[j] = start_write(j)
        for j in range(max(0, n_chunks - _NBUF), n_chunks):
            writes[j].wait()

    return embed


def kernel(tokens, W_E):
    d_model = W_E.shape[1]
    b = tokens.size
    assert b % (_NW * _CHUNK) == 0
    n_chunks = b // (_NW * _CHUNK)
    idx = tokens.reshape(_NW, n_chunks, _CHUNK).astype(jnp.int32)
    out = _build_embed(W_E.shape[0], d_model, n_chunks)(idx, W_E)
    return out.reshape(*tokens.shape, d_model)
